# baseline (speedup 1.0000x reference)
import jax
import jax.numpy as jnp
from jax import lax
from jax.experimental import pallas as pl
from jax.experimental.pallas import tpu as pltpu
from jax.experimental.pallas import tpu_sc as plsc

VOCAB = 1000000
D = 64
PMAX = 200
B = 4096
S = 200
NC, NS = 2, 16
NW = NC * NS
T = 4
G = B // (128 * T)
UNITS = S * G
UNITS_W = UNITS // NW
LANES = 16
DB = D // 8


def _unit(u):
    return u // G, u % G


def _body(seqT_hbm, pidT_hbm, tok_hbm, pos_hbm, out_hbm,
          idx_a, pidx_a, idx_b, pidx_b, buf_a, buf_b, tb, pos_sp,
          sem_ta, sem_tb, sem_pa, sem_pb, sem_w):
    c = lax.axis_index("c")
    sub = lax.axis_index("s")
    wid = sub * NC + c
    base = wid * UNITS_W
    end = base + UNITS_W

    @pl.when(sub == 0)
    def _stage():
        pltpu.sync_copy(pos_hbm, pos_sp)

    plsc.subcore_barrier()

    lane = lax.iota(jnp.int32, LANES)
    rotv = [(lane + k) & (LANES - 1) for k in range(LANES)]
    stv = [rotv[k] * 128 + lane for k in range(LANES)]

    def load_idx(u, idx_v, pidx_v, buf, sem_p):
        s_idx, g = _unit(u)
        pltpu.sync_copy(seqT_hbm.at[s_idx, g], idx_v)
        pltpu.sync_copy(pidT_hbm.at[s_idx, g], pidx_v)
        for t in range(T):
            pltpu.async_copy(pos_sp.at[pidx_v.at[t]], buf.at[t], sem_p)

    def start_tok(idx_v, pidx_v, buf, sem_p, sem_t):
        for t in range(T):
            pltpu.make_async_copy(pos_sp.at[pidx_v.at[t]], buf.at[t],
                                  sem_p).wait()
        for t in range(T):
            pltpu.async_copy(tok_hbm.at[idx_v.at[t]], buf.at[t], sem_t,
                             add=True)

    def wait_tok(idx_v, buf, sem_t):
        for t in range(T):
            pltpu.make_async_copy(tok_hbm.at[idx_v.at[t]], buf.at[t],
                                  sem_t).wait()

    def transpose(buf, tb):
        for t in range(T):
            bt = buf.at[t]
            tt = tb.at[t]

            def block_body(m, acc):
                g16 = (m % 8) * LANES
                j16 = (m // 8) * LANES
                row = g16 + lane
                sb = j16 * 128 + g16
                for k in range(LANES):
                    v = plsc.load_gather(bt, [row, rotv[k] + j16])
                    plsc.store_scatter(tt, [stv[k] + sb], v)
                return acc

            lax.fori_loop(0, 32, block_body, 0, unroll=2)

    def drain_writes(u, tb, sem_w):
        s_idx, g = _unit(u)
        for db in range(DB):
            pltpu.make_async_copy(tb.at[:, pl.ds(db * 1024, 1024)],
                                  out_hbm.at[s_idx, db, g], sem_w).wait()

    def write_unit(u, tb, sem_w):
        s_idx, g = _unit(u)
        for db in range(DB):
            pltpu.async_copy(tb.at[:, pl.ds(db * 1024, 1024)],
                             out_hbm.at[s_idx, db, g], sem_w)

    slot_a = (idx_a, pidx_a, buf_a, sem_ta, sem_pa)
    slot_b = (idx_b, pidx_b, buf_b, sem_tb, sem_pb)

    load_idx(base, idx_a, pidx_a, buf_a, sem_pa)
    start_tok(idx_a, pidx_a, buf_a, sem_pa, sem_ta)
    load_idx(base + 1, idx_b, pidx_b, buf_b, sem_pb)

    def phase(u, cur, nxt):
        idx_c, pidx_c, buf_c, sem_tc, sem_pc = cur
        idx_n, pidx_n, buf_n, sem_tn, sem_pn = nxt
        @pl.when(u >= base + 1)
        def _drain():
            drain_writes(u - 1, tb, sem_w)
        wait_tok(idx_c, buf_c, sem_tc)

        @pl.when(u + 1 < end)
        def _tok_next():
            start_tok(idx_n, pidx_n, buf_n, sem_pn, sem_tn)
        transpose(buf_c, tb)

        @pl.when(u + 2 < end)
        def _stage_next():
            load_idx(u + 2, idx_c, pidx_c, buf_c, sem_pc)
        write_unit(u, tb, sem_w)

    def pair_body(i, carry):
        u_a = base + 2 * i
        phase(u_a, slot_a, slot_b)
        phase(u_a + 1, slot_b, slot_a)
        return carry

    lax.fori_loop(0, UNITS_W // 2, pair_body, 0, unroll=False)
    drain_writes(end - 1, tb, sem_w)


@jax.jit
def _embed_sum(seqT, pidT, token_table, pos_table):
    mesh = plsc.VectorSubcoreMesh(core_axis_name="c", subcore_axis_name="s")
    kern = pl.kernel(
        _body,
        out_type=jax.ShapeDtypeStruct((S, DB, G, T, 1024), jnp.float32),
        mesh=mesh,
        scratch_types=[
            pltpu.VMEM((T, 128), jnp.int32),
            pltpu.VMEM((T, 128), jnp.int32),
            pltpu.VMEM((T, 128), jnp.int32),
            pltpu.VMEM((T, 128), jnp.int32),
            pltpu.VMEM((T, 128, D), jnp.float32),
            pltpu.VMEM((T, 128, D), jnp.float32),
            pltpu.VMEM((T, D * 128), jnp.float32),
            pltpu.VMEM_SHARED((PMAX, D), jnp.float32),
            pltpu.SemaphoreType.DMA,
            pltpu.SemaphoreType.DMA,
            pltpu.SemaphoreType.DMA,
            pltpu.SemaphoreType.DMA,
            pltpu.SemaphoreType.DMA,
        ],
        compiler_params=pltpu.CompilerParams(use_tc_tiling_on_sc=False,
                                             needs_layout_passes=False),
    )
    return kern(seqT, pidT, token_table, pos_table)


def kernel(sequence, position_ids, token_table, pos_table):
    tok2 = jnp.pad(token_table, ((0, 0), (0, D))).reshape(2 * VOCAB, D)
    seqT = (sequence.T * 2).astype(jnp.int32).reshape(S, G, T, 128)
    pidT = position_ids.T.astype(jnp.int32).reshape(S, G, T, 128)
    w = _embed_sum(seqT, pidT, tok2, pos_table)
    x = (w.reshape(S, DB, G, T, 8, 128)
          .transpose(0, 1, 4, 2, 3, 5)
          .reshape(S, D, B))
    return x.transpose(2, 0, 1)
